# Initial kernel scaffold; baseline (speedup 1.0000x reference)
import functools

import jax
import jax.numpy as jnp
from jax import lax
from jax.experimental import pallas as pl
from jax.experimental.pallas import tpu as pltpu
from jax.experimental.pallas import tpu_sc as plsc

N = 10000
D = 128
E = 320000
L = 8

NC = 2
NS = 16
NW = NC * NS
CHUNK = 128
NCH = (E + NW * CHUNK - 1) // (NW * CHUNK)
PE = NW * NCH * CHUNK
AGGR = 10240
PAD_ROW = N
ROWS_PER_TILE = AGGR // NS

_BN_INV = 1.0 / (1.0 + 1e-5) ** 0.5


@functools.partial(
    pl.kernel,
    out_type=jax.ShapeDtypeStruct((NC * AGGR, D), jnp.float32),
    mesh=plsc.VectorSubcoreMesh(core_axis_name="c", subcore_axis_name="s"),
    scratch_types=[
        pltpu.VMEM((NCH, CHUNK), jnp.int32),
        pltpu.VMEM((NCH, CHUNK), jnp.int32),
        pltpu.VMEM((CHUNK, D), jnp.float32),
        pltpu.VMEM_SHARED((AGGR, D), jnp.float32),
        pltpu.SemaphoreType.DMA,
    ],
)
def _sc_agg(h_hbm, src_hbm, dst_hbm, out_hbm, src_v, dst_v, rows_v, agg_sh, sem):
    cid = lax.axis_index("c")
    sid = lax.axis_index("s")
    wid = sid * NC + cid

    def _zrow(i, carry):
        zv = jnp.zeros((16,), jnp.float32)
        for j in range(D // 16):
            rows_v[i, pl.ds(j * 16, 16)] = zv
        return carry

    lax.fori_loop(0, CHUNK, _zrow, 0)
    for k in range(ROWS_PER_TILE // CHUNK):
        pltpu.sync_copy(rows_v, agg_sh.at[pl.ds(sid * ROWS_PER_TILE + k * CHUNK, CHUNK)])
    plsc.subcore_barrier()

    pltpu.sync_copy(src_hbm.at[wid], src_v)
    pltpu.sync_copy(dst_hbm.at[wid], dst_v)

    def _edge_chunk(j, carry):
        pltpu.async_copy(h_hbm.at[src_v.at[j]], rows_v, sem).wait()
        pltpu.sync_copy(rows_v, agg_sh.at[dst_v.at[j]], add=True)
        return carry

    lax.fori_loop(0, NCH, _edge_chunk, 0)
    plsc.subcore_barrier()

    pltpu.sync_copy(
        agg_sh.at[pl.ds(sid * ROWS_PER_TILE, ROWS_PER_TILE)],
        out_hbm.at[pl.ds(cid * AGGR + sid * ROWS_PER_TILE, ROWS_PER_TILE)],
    )


_BLK = 1000


def _mlp_body(eps_ref, h_ref, p_ref, w1_ref, b1_ref, g1_ref, be1_ref,
              w2_ref, b2_ref, g2_ref, be2_ref, out_ref):
    agg = p_ref[0] + p_ref[1]
    m = (1.0 + eps_ref[0, 0]) * h_ref[...] + agg
    t = jnp.dot(m, w1_ref[...], preferred_element_type=jnp.float32) + b1_ref[...]
    t = jnp.maximum(t * (g1_ref[...] * _BN_INV) + be1_ref[...], 0.0)
    u = jnp.dot(t, w2_ref[...], preferred_element_type=jnp.float32) + b2_ref[...]
    out_ref[...] = jnp.maximum(u * (g2_ref[...] * _BN_INV) + be2_ref[...], 0.0)


def _mlp_layer(eps_i, h, parts, w1, b1, g1, be1, w2, b2, g2, be2):
    full = lambda shape: pl.BlockSpec(shape, lambda i: (0,) * len(shape))
    return pl.pallas_call(
        _mlp_body,
        grid=(N // _BLK,),
        in_specs=[
            pl.BlockSpec(memory_space=pltpu.SMEM),
            pl.BlockSpec((_BLK, D), lambda i: (i, 0)),
            pl.BlockSpec((NC, _BLK, D), lambda i: (0, i, 0)),
            full((D, D)), full((1, D)), full((1, D)), full((1, D)),
            full((D, D)), full((1, D)), full((1, D)), full((1, D)),
        ],
        out_specs=pl.BlockSpec((_BLK, D), lambda i: (i, 0)),
        out_shape=jax.ShapeDtypeStruct((N, D), jnp.float32),
        compiler_params=pltpu.CompilerParams(
            dimension_semantics=("arbitrary",),
        ),
    )(eps_i, h, parts, w1, b1, g1, be1, w2, b2, g2, be2)


def _reg_body(s0_ref, s1_ref, s2_ref, s3_ref, wr1_ref, br1_ref, gr_ref,
              ber_ref, wr2_ref, br2_ref, out_ref):
    h = (s0_ref[...] + s1_ref[...] + s2_ref[...] + s3_ref[...]) * 0.25
    r = jnp.dot(h, wr1_ref[...], preferred_element_type=jnp.float32) + br1_ref[...]
    r = jnp.maximum(r * (gr_ref[...] * _BN_INV) + ber_ref[...], 0.0)
    out_ref[...] = jnp.dot(r, wr2_ref[...], preferred_element_type=jnp.float32) + br2_ref[...]


def _regressor(s0, s1, s2, s3, wr1, br1, gr, ber, wr2p, br2p):
    full = lambda shape: pl.BlockSpec(shape, lambda i: (0,) * len(shape))
    return pl.pallas_call(
        _reg_body,
        grid=(N // _BLK,),
        in_specs=[
            pl.BlockSpec((_BLK, D), lambda i: (i, 0)),
            pl.BlockSpec((_BLK, D), lambda i: (i, 0)),
            pl.BlockSpec((_BLK, D), lambda i: (i, 0)),
            pl.BlockSpec((_BLK, D), lambda i: (i, 0)),
            full((D, D)), full((1, D)), full((1, D)), full((1, D)),
            full((D, D)), full((1, D)),
        ],
        out_specs=pl.BlockSpec((_BLK, D), lambda i: (i, 0)),
        out_shape=jax.ShapeDtypeStruct((N, D), jnp.float32),
        compiler_params=pltpu.CompilerParams(
            dimension_semantics=("arbitrary",),
        ),
    )(s0, s1, s2, s3, wr1, br1, gr, ber, wr2p, br2p)


def _aggregate(h, src3, dst3):
    parts = _sc_agg(h, src3, dst3)
    return parts.reshape(NC, AGGR, D)


def kernel(x, edge_index, eps, W1, b1, g1, be1, W2, b2, g2, be2,
           Wr1, br1, gr, ber, Wr2, br2):
    src = edge_index[0].astype(jnp.int32)
    dst = edge_index[1].astype(jnp.int32)
    src3 = jnp.concatenate(
        [src, jnp.zeros((PE - E,), jnp.int32)]).reshape(NW, NCH, CHUNK)
    dst3 = jnp.concatenate(
        [dst, jnp.full((PE - E,), PAD_ROW, jnp.int32)]).reshape(NW, NCH, CHUNK)

    eps2 = eps.reshape(L, 1, 1)
    b1r = b1.reshape(L, 1, D)
    g1r = g1.reshape(L, 1, D)
    be1r = be1.reshape(L, 1, D)
    b2r = b2.reshape(L, 1, D)
    g2r = g2.reshape(L, 1, D)
    be2r = be2.reshape(L, 1, D)

    h = x
    skips = []
    for i in range(L):
        parts = _aggregate(h, src3, dst3)
        h = _mlp_layer(eps2[i], h, parts, W1[i], b1r[i], g1r[i], be1r[i],
                       W2[i], b2r[i], g2r[i], be2r[i])
        if i % 2 == 1:
            skips.append(h)

    wr2p = jnp.zeros((D, D), jnp.float32).at[:, :1].set(Wr2)
    br2p = jnp.zeros((1, D), jnp.float32).at[0, 0].set(br2[0])
    out_full = _regressor(skips[0], skips[1], skips[2], skips[3],
                          Wr1, br1.reshape(1, D), gr.reshape(1, D),
                          ber.reshape(1, D), wr2p, br2p)
    return out_full[:, :1]


# speedup vs baseline: 3.8033x; 3.8033x over previous
import functools

import jax
import jax.numpy as jnp
from jax import lax
from jax.experimental import pallas as pl
from jax.experimental.pallas import tpu as pltpu
from jax.experimental.pallas import tpu_sc as plsc

N = 10000
D = 128
E = 320000
L = 8

NC = 2
NS = 16
NW = NC * NS
CHUNK = 128
NCH = (E + NW * CHUNK - 1) // (NW * CHUNK)
PE = NW * NCH * CHUNK
AGGR = 10240
PAD_ROW = N
ROWS_PER_TILE = AGGR // NS

_BN_INV = 1.0 / (1.0 + 1e-5) ** 0.5


@functools.cache
def _make_sc_agg():
    return functools.partial(
        pl.kernel,
        out_type=jax.ShapeDtypeStruct((NC * AGGR, D), jnp.float32),
        mesh=plsc.VectorSubcoreMesh(core_axis_name="c", subcore_axis_name="s"),
        scratch_types=[
            pltpu.VMEM((NCH, CHUNK), jnp.int32),
            pltpu.VMEM((NCH, CHUNK), jnp.int32),
            pltpu.VMEM((CHUNK, D), jnp.float32),
            pltpu.VMEM_SHARED((AGGR, D), jnp.float32),
            pltpu.SemaphoreType.DMA,
        ],
    )(_sc_agg_body)


def _sc_agg_body(h_hbm, src_hbm, dst_hbm, out_hbm, src_v, dst_v, rows_v, agg_sh, sem):
    cid = lax.axis_index("c")
    sid = lax.axis_index("s")
    wid = sid * NC + cid

    def _zrow(i, carry):
        zv = jnp.zeros((16,), jnp.float32)
        for j in range(D // 16):
            rows_v[i, pl.ds(j * 16, 16)] = zv
        return carry

    lax.fori_loop(0, CHUNK, _zrow, 0)
    for k in range(ROWS_PER_TILE // CHUNK):
        pltpu.sync_copy(rows_v, agg_sh.at[pl.ds(sid * ROWS_PER_TILE + k * CHUNK, CHUNK)])
    plsc.subcore_barrier()

    pltpu.sync_copy(src_hbm.at[wid], src_v)
    pltpu.sync_copy(dst_hbm.at[wid], dst_v)

    def _edge_chunk(j, carry):
        pltpu.async_copy(h_hbm.at[src_v.at[j]], rows_v, sem).wait()
        pltpu.sync_copy(rows_v, agg_sh.at[dst_v.at[j]], add=True)
        return carry

    lax.fori_loop(0, NCH, _edge_chunk, 0)
    plsc.subcore_barrier()

    pltpu.sync_copy(
        agg_sh.at[pl.ds(sid * ROWS_PER_TILE, ROWS_PER_TILE)],
        out_hbm.at[pl.ds(cid * AGGR + sid * ROWS_PER_TILE, ROWS_PER_TILE)],
    )


_BLK = 1000


def _mlp_body(eps_ref, h_ref, p_ref, w1_ref, b1_ref, g1_ref, be1_ref,
              w2_ref, b2_ref, g2_ref, be2_ref, out_ref):
    agg = p_ref[0] + p_ref[1]
    m = (1.0 + eps_ref[0, 0]) * h_ref[...] + agg
    t = jnp.dot(m, w1_ref[...], preferred_element_type=jnp.float32) + b1_ref[...]
    t = jnp.maximum(t * (g1_ref[...] * _BN_INV) + be1_ref[...], 0.0)
    u = jnp.dot(t, w2_ref[...], preferred_element_type=jnp.float32) + b2_ref[...]
    out_ref[...] = jnp.maximum(u * (g2_ref[...] * _BN_INV) + be2_ref[...], 0.0)


def _mlp_layer(eps_i, h, parts, w1, b1, g1, be1, w2, b2, g2, be2):
    full = lambda shape: pl.BlockSpec(shape, lambda i: (0,) * len(shape))
    return pl.pallas_call(
        _mlp_body,
        grid=(N // _BLK,),
        in_specs=[
            pl.BlockSpec(memory_space=pltpu.SMEM),
            pl.BlockSpec((_BLK, D), lambda i: (i, 0)),
            pl.BlockSpec((NC, _BLK, D), lambda i: (0, i, 0)),
            full((D, D)), full((1, D)), full((1, D)), full((1, D)),
            full((D, D)), full((1, D)), full((1, D)), full((1, D)),
        ],
        out_specs=pl.BlockSpec((_BLK, D), lambda i: (i, 0)),
        out_shape=jax.ShapeDtypeStruct((N, D), jnp.float32),
        compiler_params=pltpu.CompilerParams(
            dimension_semantics=("arbitrary",),
        ),
    )(eps_i, h, parts, w1, b1, g1, be1, w2, b2, g2, be2)


def _reg_body(s0_ref, s1_ref, s2_ref, s3_ref, wr1_ref, br1_ref, gr_ref,
              ber_ref, wr2_ref, br2_ref, out_ref):
    h = (s0_ref[...] + s1_ref[...] + s2_ref[...] + s3_ref[...]) * 0.25
    r = jnp.dot(h, wr1_ref[...], preferred_element_type=jnp.float32) + br1_ref[...]
    r = jnp.maximum(r * (gr_ref[...] * _BN_INV) + ber_ref[...], 0.0)
    out_ref[...] = jnp.dot(r, wr2_ref[...], preferred_element_type=jnp.float32) + br2_ref[...]


def _regressor(s0, s1, s2, s3, wr1, br1, gr, ber, wr2p, br2p):
    full = lambda shape: pl.BlockSpec(shape, lambda i: (0,) * len(shape))
    return pl.pallas_call(
        _reg_body,
        grid=(N // _BLK,),
        in_specs=[
            pl.BlockSpec((_BLK, D), lambda i: (i, 0)),
            pl.BlockSpec((_BLK, D), lambda i: (i, 0)),
            pl.BlockSpec((_BLK, D), lambda i: (i, 0)),
            pl.BlockSpec((_BLK, D), lambda i: (i, 0)),
            full((D, D)), full((1, D)), full((1, D)), full((1, D)),
            full((D, D)), full((1, D)),
        ],
        out_specs=pl.BlockSpec((_BLK, D), lambda i: (i, 0)),
        out_shape=jax.ShapeDtypeStruct((N, D), jnp.float32),
        compiler_params=pltpu.CompilerParams(
            dimension_semantics=("arbitrary",),
        ),
    )(s0, s1, s2, s3, wr1, br1, gr, ber, wr2p, br2p)


def _aggregate(h, src3, dst3):
    parts = _make_sc_agg()(h, src3, dst3)
    return parts.reshape(NC, AGGR, D)


def kernel(x, edge_index, eps, W1, b1, g1, be1, W2, b2, g2, be2,
           Wr1, br1, gr, ber, Wr2, br2):
    src = edge_index[0].astype(jnp.int32)
    dst = edge_index[1].astype(jnp.int32)
    src3 = jnp.concatenate(
        [src, jnp.zeros((PE - E,), jnp.int32)]).reshape(NW, NCH, CHUNK)
    dst3 = jnp.concatenate(
        [dst, jnp.full((PE - E,), PAD_ROW, jnp.int32)]).reshape(NW, NCH, CHUNK)

    eps2 = eps.reshape(L, 1, 1)
    b1r = b1.reshape(L, 1, D)
    g1r = g1.reshape(L, 1, D)
    be1r = be1.reshape(L, 1, D)
    b2r = b2.reshape(L, 1, D)
    g2r = g2.reshape(L, 1, D)
    be2r = be2.reshape(L, 1, D)

    h = x
    skips = []
    for i in range(L):
        parts = _aggregate(h, src3, dst3)
        h = _mlp_layer(eps2[i], h, parts, W1[i], b1r[i], g1r[i], be1r[i],
                       W2[i], b2r[i], g2r[i], be2r[i])
        if i % 2 == 1:
            skips.append(h)

    wr2p = jnp.zeros((D, D), jnp.float32).at[:, :1].set(Wr2)
    br2p = jnp.zeros((1, D), jnp.float32).at[0, 0].set(br2[0])
    out_full = _regressor(skips[0], skips[1], skips[2], skips[3],
                          Wr1, br1.reshape(1, D), gr.reshape(1, D),
                          ber.reshape(1, D), wr2p, br2p)
    return out_full[:, :1]


# speedup vs baseline: 6.1416x; 1.6148x over previous
import functools

import jax
import jax.numpy as jnp
from jax import lax
from jax.experimental import pallas as pl
from jax.experimental.pallas import tpu as pltpu
from jax.experimental.pallas import tpu_sc as plsc

N = 10000
D = 128
E = 320000
L = 8

NC = 2
NS = 16
NW = NC * NS
CHUNK = 80
NBUF = 3
NCH = 126
PE = NW * NCH * CHUNK
AGGR = 10112
PAD_ROW = N
ROWS_PER_TILE = AGGR // NS
IDX_BITS = 14

_BN_INV = 1.0 / (1.0 + 1e-5) ** 0.5


@functools.cache
def _make_sc_agg():
    return functools.partial(
        pl.kernel,
        out_type=jax.ShapeDtypeStruct((NC * AGGR, D), jnp.float32),
        mesh=plsc.VectorSubcoreMesh(core_axis_name="c", subcore_axis_name="s"),
        scratch_types=[
            pltpu.VMEM((NCH * CHUNK,), jnp.int32),
            pltpu.VMEM((8, CHUNK), jnp.int32),
            pltpu.VMEM((8, CHUNK), jnp.int32),
            pltpu.VMEM((NBUF, CHUNK, D), jnp.float32),
            pltpu.VMEM_SHARED((AGGR, D), jnp.float32),
        ]
        + [pltpu.SemaphoreType.DMA] * NBUF,
    )(_sc_agg_body)


def _sc_agg_body(h_hbm, pk_hbm, out_hbm, pk_v, src_v, dst_v, buf, agg_sh,
                 *sems):
    gs = sems
    cid = lax.axis_index("c")
    sid = lax.axis_index("s")
    wid = sid * NC + cid

    def _zrow(i, carry):
        zv = jnp.zeros((16,), jnp.float32)
        for j in range(D // 16):
            buf[0, i, pl.ds(j * 16, 16)] = zv
        return carry

    lax.fori_loop(0, CHUNK, _zrow, 0)
    for k in range(ROWS_PER_TILE // CHUNK):
        pltpu.sync_copy(buf.at[0],
                        agg_sh.at[pl.ds(sid * ROWS_PER_TILE + k * CHUNK, CHUNK)])
    _rem = ROWS_PER_TILE - (ROWS_PER_TILE // CHUNK) * CHUNK
    if _rem:
        pltpu.sync_copy(
            buf.at[0, pl.ds(0, _rem)],
            agg_sh.at[pl.ds(sid * ROWS_PER_TILE + ROWS_PER_TILE - _rem, _rem)])
    plsc.subcore_barrier()

    pltpu.sync_copy(pk_hbm.at[wid], pk_v)

    def _unpack(j, b):
        for k in range(CHUNK // 16):
            p = pk_v[pl.ds(j * CHUNK + k * 16, 16)]
            src_v[b, pl.ds(k * 16, 16)] = p >> IDX_BITS
            dst_v[b, pl.ds(k * 16, 16)] = p & ((1 << IDX_BITS) - 1)

    def _gather(j, b):
        pltpu.async_copy(h_hbm.at[src_v.at[b]], buf.at[b], gs[b])

    def _gather_wait(j, b):
        pltpu.make_async_copy(h_hbm.at[src_v.at[b]], buf.at[b], gs[b]).wait()

    _unpack(0, 0)
    _unpack(1, 1)
    _gather(0, 0)
    _gather(1, 1)

    def _group(g, carry):
        for b in range(NBUF):
            j = g * NBUF + b
            b2 = (b + 2) % NBUF
            _gather_wait(j, b)

            @pl.when(j + 2 < NCH)
            def _():
                _unpack(j + 2, b2)
                _gather(j + 2, b2)

            pltpu.sync_copy(buf.at[b], agg_sh.at[dst_v.at[b]], add=True)

        return carry

    lax.fori_loop(0, NCH // NBUF, _group, 0)
    plsc.subcore_barrier()

    pltpu.sync_copy(
        agg_sh.at[pl.ds(sid * ROWS_PER_TILE, ROWS_PER_TILE)],
        out_hbm.at[pl.ds(cid * AGGR + sid * ROWS_PER_TILE, ROWS_PER_TILE)],
    )


_BLK = 1000


def _mlp_body(eps_ref, h_ref, p_ref, w1_ref, b1_ref, g1_ref, be1_ref,
              w2_ref, b2_ref, g2_ref, be2_ref, out_ref):
    agg = p_ref[0] + p_ref[1]
    m = (1.0 + eps_ref[0, 0]) * h_ref[...] + agg
    t = jnp.dot(m, w1_ref[...], preferred_element_type=jnp.float32) + b1_ref[...]
    t = jnp.maximum(t * (g1_ref[...] * _BN_INV) + be1_ref[...], 0.0)
    u = jnp.dot(t, w2_ref[...], preferred_element_type=jnp.float32) + b2_ref[...]
    out_ref[...] = jnp.maximum(u * (g2_ref[...] * _BN_INV) + be2_ref[...], 0.0)


def _mlp_layer(eps_i, h, parts, w1, b1, g1, be1, w2, b2, g2, be2):
    full = lambda shape: pl.BlockSpec(shape, lambda i: (0,) * len(shape))
    return pl.pallas_call(
        _mlp_body,
        grid=(N // _BLK,),
        in_specs=[
            pl.BlockSpec(memory_space=pltpu.SMEM),
            pl.BlockSpec((_BLK, D), lambda i: (i, 0)),
            pl.BlockSpec((NC, _BLK, D), lambda i: (0, i, 0)),
            full((D, D)), full((1, D)), full((1, D)), full((1, D)),
            full((D, D)), full((1, D)), full((1, D)), full((1, D)),
        ],
        out_specs=pl.BlockSpec((_BLK, D), lambda i: (i, 0)),
        out_shape=jax.ShapeDtypeStruct((N, D), jnp.float32),
        compiler_params=pltpu.CompilerParams(
            dimension_semantics=("arbitrary",),
        ),
    )(eps_i, h, parts, w1, b1, g1, be1, w2, b2, g2, be2)


def _reg_body(s0_ref, s1_ref, s2_ref, s3_ref, wr1_ref, br1_ref, gr_ref,
              ber_ref, wr2_ref, br2_ref, out_ref):
    h = (s0_ref[...] + s1_ref[...] + s2_ref[...] + s3_ref[...]) * 0.25
    r = jnp.dot(h, wr1_ref[...], preferred_element_type=jnp.float32) + br1_ref[...]
    r = jnp.maximum(r * (gr_ref[...] * _BN_INV) + ber_ref[...], 0.0)
    out_ref[...] = jnp.dot(r, wr2_ref[...], preferred_element_type=jnp.float32) + br2_ref[...]


def _regressor(s0, s1, s2, s3, wr1, br1, gr, ber, wr2p, br2p):
    full = lambda shape: pl.BlockSpec(shape, lambda i: (0,) * len(shape))
    return pl.pallas_call(
        _reg_body,
        grid=(N // _BLK,),
        in_specs=[
            pl.BlockSpec((_BLK, D), lambda i: (i, 0)),
            pl.BlockSpec((_BLK, D), lambda i: (i, 0)),
            pl.BlockSpec((_BLK, D), lambda i: (i, 0)),
            pl.BlockSpec((_BLK, D), lambda i: (i, 0)),
            full((D, D)), full((1, D)), full((1, D)), full((1, D)),
            full((D, D)), full((1, D)),
        ],
        out_specs=pl.BlockSpec((_BLK, D), lambda i: (i, 0)),
        out_shape=jax.ShapeDtypeStruct((N, D), jnp.float32),
        compiler_params=pltpu.CompilerParams(
            dimension_semantics=("arbitrary",),
        ),
    )(s0, s1, s2, s3, wr1, br1, gr, ber, wr2p, br2p)


def _aggregate(h, pk2):
    parts = _make_sc_agg()(h, pk2)
    return parts.reshape(NC, AGGR, D)


def kernel(x, edge_index, eps, W1, b1, g1, be1, W2, b2, g2, be2,
           Wr1, br1, gr, ber, Wr2, br2):
    src = edge_index[0].astype(jnp.int32)
    dst = edge_index[1].astype(jnp.int32)
    packed = (src << IDX_BITS) | dst
    pk2 = jnp.concatenate(
        [packed, jnp.full((PE - E,), PAD_ROW, jnp.int32)]).reshape(NW, NCH * CHUNK)

    eps2 = eps.reshape(L, 1, 1)
    b1r = b1.reshape(L, 1, D)
    g1r = g1.reshape(L, 1, D)
    be1r = be1.reshape(L, 1, D)
    b2r = b2.reshape(L, 1, D)
    g2r = g2.reshape(L, 1, D)
    be2r = be2.reshape(L, 1, D)

    h = x
    skips = []
    for i in range(L):
        parts = _aggregate(h, pk2)
        h = _mlp_layer(eps2[i], h, parts, W1[i], b1r[i], g1r[i], be1r[i],
                       W2[i], b2r[i], g2r[i], be2r[i])
        if i % 2 == 1:
            skips.append(h)

    wr2p = jnp.zeros((D, D), jnp.float32).at[:, :1].set(Wr2)
    br2p = jnp.zeros((1, D), jnp.float32).at[0, 0].set(br2[0])
    out_full = _regressor(skips[0], skips[1], skips[2], skips[3],
                          Wr1, br1.reshape(1, D), gr.reshape(1, D),
                          ber.reshape(1, D), wr2p, br2p)
    return out_full[:, :1]
